# Initial kernel scaffold; baseline (speedup 1.0000x reference)
import functools

import jax
import jax.numpy as jnp
from jax import lax
from jax.experimental import pallas as pl
from jax.experimental.pallas import tpu as pltpu
from jax.experimental.pallas import tpu_sc as plsc

K = 16
B = 1024
D_IN = 512
D_REP = 128
N = 100000
BN = 2048
N_PAD = 102400
NBLK = N_PAD // BN
C16 = N_PAD // 16
C256 = N_PAD // 256
NW = 32
RPW = B // NW
PAD_VAL = 1e4
INF = float(3e38)


def _tc_body(x_ref, w_ref, reps_ref, d2_ref, cm16_ref, cm256_ref, rt_ref):
    nb = pl.program_id(0)

    @pl.when(nb == 0)
    def _():
        rt_ref[...] = lax.dot_general(
            w_ref[...], x_ref[...],
            dimension_numbers=(((0,), (1,)), ((), ())),
            preferred_element_type=jnp.float32)

    rt = rt_ref[...]
    q2 = jnp.sum(rt * rt, axis=0, keepdims=True)
    reps = reps_ref[...]
    r2 = jnp.sum(reps * reps, axis=1, keepdims=True)
    cross = lax.dot_general(
        reps, rt, dimension_numbers=(((1,), (0,)), ((), ())),
        preferred_element_type=jnp.float32)
    d2 = r2 + q2 - 2.0 * cross
    d2_ref[...] = d2
    m16 = jnp.min(d2.reshape(BN // 16, 16, B), axis=1)
    cm16_ref[...] = m16
    cm256_ref[...] = jnp.min(m16.reshape(BN // 256, 16, B), axis=1)


_tc_call = pl.pallas_call(
    _tc_body,
    grid=(NBLK,),
    in_specs=[
        pl.BlockSpec((B, D_IN), lambda i: (0, 0)),
        pl.BlockSpec((D_IN, D_REP), lambda i: (0, 0)),
        pl.BlockSpec((BN, D_REP), lambda i: (i, 0)),
    ],
    out_specs=[
        pl.BlockSpec((BN, B), lambda i: (i, 0)),
        pl.BlockSpec((BN // 16, B), lambda i: (i, 0)),
        pl.BlockSpec((BN // 256, B), lambda i: (i, 0)),
    ],
    out_shape=[
        jax.ShapeDtypeStruct((N_PAD, B), jnp.float32),
        jax.ShapeDtypeStruct((C16, B), jnp.float32),
        jax.ShapeDtypeStruct((C256, B), jnp.float32),
    ],
    scratch_shapes=[pltpu.VMEM((D_REP, B), jnp.float32)],
)


def _full16(v):
    return jnp.full((16,), v, jnp.int32)


def _merge16(bv, bi, nv, ni):
    sv, si = plsc.sort_key_val(nv, ni)
    rv = lax.rev(sv, (0,))
    ri = lax.rev(si, (0,))
    take = bv <= rv
    lov = jnp.where(take, bv, rv)
    loi = jnp.where(take, bi, ri)
    return plsc.sort_key_val(lov, loi)


def _scan_topk(nvec, load_fn):
    def body(i, carry):
        bv, bi, thr = carry
        nv, ni = load_fn(i)
        vmin = jnp.min(nv)

        def do(_):
            mv, mi = _merge16(bv, bi, nv, ni)
            return mv, mi, jnp.max(mv)

        def skip(_):
            return bv, bi, thr

        return lax.cond(vmin < thr, do, skip, None)

    bv0 = jnp.full((16,), INF, jnp.float32)
    bi0 = jnp.zeros((16,), jnp.int32)
    bv, bi, _ = lax.fori_loop(0, nvec, body, (bv0, bi0, jnp.float32(INF)))
    return bv, bi


def _rsqrt_newton(x):
    i = plsc.bitcast(x, jnp.int32)
    i = jnp.int32(0x5F3759DF) - lax.shift_right_arithmetic(i, jnp.int32(1))
    y = plsc.bitcast(i, jnp.float32)
    for _ in range(3):
        y = y * (1.5 - 0.5 * x * y * y)
    return y


def _sc_body(cm256_hbm, cm16_hbm, d2_hbm, act_hbm, out_hbm,
             slab, cidb, fidb, gidx, cand1, cand2, aidx, acand, wbuf, outb,
             sem):
    cid = lax.axis_index("core")
    sid = lax.axis_index("subcore")
    wid = cid * 16 + sid
    base = wid * RPW
    iota = lax.iota(jnp.int32, 16)

    pltpu.sync_copy(cm256_hbm.at[:, pl.ds(base, RPW)], slab)

    @pl.loop(0, RPW)
    def _(r):
        q = base + r

        def load_coarse(i):
            rows = i * 16 + iota
            v = plsc.load_gather(slab, [rows, _full16(r)])
            return v, rows

        _, ci = _scan_topk(C256 // 16, load_coarse)
        plsc.store_scatter(cidb, [_full16(r), iota], ci)

        @pl.loop(0, K)
        def _(j):
            cj = plsc.load_gather(cidb, [_full16(r), _full16(j)])
            idxv = (cj * 16 + iota) * B + q
            gidx[pl.ds(r * 256 + j * 16, 16)] = idxv

    @pl.loop(0, 8)
    def _(g):
        cs = []
        for u in range(8):
            off = (g * 8 + u) * 128
            cs.append(pltpu.async_copy(
                cm16_hbm.at[gidx.at[pl.ds(off, 128)]],
                cand1.at[pl.ds(off, 128)], sem))
        for c in cs:
            c.wait()

    @pl.loop(0, RPW)
    def _(r):
        q = base + r

        def load_fine(j):
            v = cand1[pl.ds(r * 256 + j * 16, 16)]
            cj = plsc.load_gather(cidb, [_full16(r), _full16(j)])
            return v, cj * 16 + iota

        _, fi = _scan_topk(K, load_fine)
        plsc.store_scatter(fidb, [_full16(r), iota], fi)

        @pl.loop(0, K)
        def _(j):
            fj = plsc.load_gather(fidb, [_full16(r), _full16(j)])
            idxv = (fj * 16 + iota) * B + q
            gidx[pl.ds(r * 256 + j * 16, 16)] = idxv

    @pl.loop(0, 8)
    def _(g):
        cs = []
        for u in range(8):
            off = (g * 8 + u) * 128
            cs.append(pltpu.async_copy(
                d2_hbm.at[gidx.at[pl.ds(off, 128)]],
                cand2.at[pl.ds(off, 128)], sem))
        for c in cs:
            c.wait()

    @pl.loop(0, RPW)
    def _(r):
        def load_elem(j):
            v = cand2[pl.ds(r * 256 + j * 16, 16)]
            fj = plsc.load_gather(fidb, [_full16(r), _full16(j)])
            return v, fj * 16 + iota

        dv, di = _scan_topk(K, load_elem)
        d2c = jnp.maximum(dv, 1e-12)
        d = d2c * _rsqrt_newton(d2c)
        dmin = jnp.min(d)
        e = jnp.exp(dmin - d)
        w = e / jnp.sum(e)
        plsc.store_scatter(wbuf, [_full16(r), iota], w)
        aidx[pl.ds(r * 16, 16)] = di

    cs = []
    for bb in range(4):
        cs.append(pltpu.async_copy(
            act_hbm.at[aidx.at[pl.ds(bb * 128, 128)]],
            acand.at[pl.ds(bb * 128, 128), :], sem))
    for c in cs:
        c.wait()

    @pl.loop(0, RPW)
    def _(r):
        def body(j, acc):
            a = plsc.load_gather(acand, [_full16(r * 16 + j), iota])
            wj = plsc.load_gather(wbuf, [_full16(r), _full16(j)])
            return acc + wj * a

        acc = lax.fori_loop(0, K, body, jnp.zeros((16,), jnp.float32))
        plsc.store_scatter(outb, [_full16(r), iota], acc)

    pltpu.sync_copy(outb, out_hbm.at[pl.ds(base, RPW)])


_sc_call = functools.partial(
    pl.kernel,
    out_type=jax.ShapeDtypeStruct((B, 16), jnp.float32),
    mesh=plsc.VectorSubcoreMesh(core_axis_name="core",
                                subcore_axis_name="subcore"),
    scratch_types=[
        pltpu.VMEM((C256, RPW), jnp.float32),
        pltpu.VMEM((RPW, 16), jnp.int32),
        pltpu.VMEM((RPW, 16), jnp.int32),
        pltpu.VMEM((RPW * 256,), jnp.int32),
        pltpu.VMEM((RPW * 256,), jnp.float32),
        pltpu.VMEM((RPW * 256,), jnp.float32),
        pltpu.VMEM((RPW * 16,), jnp.int32),
        pltpu.VMEM((RPW * 16, 16), jnp.float32),
        pltpu.VMEM((RPW, 16), jnp.float32),
        pltpu.VMEM((RPW, 16), jnp.float32),
        pltpu.SemaphoreType.DMA,
    ],
)(_sc_body)


def kernel(batch_images, W_enc, representations, actions):
    reps_p = jnp.pad(representations, ((0, N_PAD - N), (0, 0)),
                     constant_values=PAD_VAL)
    act_p = jnp.pad(actions, ((0, 0), (0, 16 - actions.shape[1])))
    d2t, cm16t, cm256t = _tc_call(batch_images, W_enc, reps_p)
    out = _sc_call(cm256t, cm16t.reshape(-1), d2t.reshape(-1), act_p)
    return out[:, :7]


# speedup vs baseline: 5.9667x; 5.9667x over previous
import dataclasses
import functools

import jax
import jax.numpy as jnp
from jax import lax
from jax.experimental import pallas as pl
from jax.experimental.pallas import tpu as pltpu
from jax.experimental.pallas import tpu_sc as plsc

K = 16
B = 1024
D_IN = 512
D_REP = 128
N = 100000
BN = 2048
N_PAD = 102400
NBLK = N_PAD // BN
C16 = N_PAD // 16
C256 = N_PAD // 256
NW = 32
RPW = B // NW
PAD_VAL = 1e4
INF = float(3e38)


def _tc_body(x_ref, w_ref, reps_ref, d2_ref, cm16_ref, cm256_ref, rt_ref):
    nb = pl.program_id(0)

    @pl.when(nb == 0)
    def _():
        rt_ref[...] = lax.dot_general(
            w_ref[...], x_ref[...],
            dimension_numbers=(((0,), (1,)), ((), ())),
            preferred_element_type=jnp.float32)

    rt = rt_ref[...]
    q2 = jnp.sum(rt * rt, axis=0, keepdims=True)
    reps = reps_ref[...]
    r2 = jnp.sum(reps * reps, axis=1, keepdims=True)
    cross = lax.dot_general(
        reps, rt, dimension_numbers=(((1,), (0,)), ((), ())),
        preferred_element_type=jnp.float32)
    d2 = r2 + q2 - 2.0 * cross
    d2_ref[...] = d2
    m16 = jnp.min(d2.reshape(BN // 16, 16, B), axis=1)
    cm16_ref[...] = m16
    cm256_ref[...] = jnp.min(m16.reshape(BN // 256, 16, B), axis=1)


_tc_call = pl.pallas_call(
    _tc_body,
    grid=(NBLK,),
    in_specs=[
        pl.BlockSpec((B, D_IN), lambda i: (0, 0)),
        pl.BlockSpec((D_IN, D_REP), lambda i: (0, 0)),
        pl.BlockSpec((BN, D_REP), lambda i: (i, 0)),
    ],
    out_specs=[
        pl.BlockSpec((BN, B), lambda i: (i, 0)),
        pl.BlockSpec((BN // 16, B), lambda i: (i, 0)),
        pl.BlockSpec((BN // 256, B), lambda i: (i, 0)),
    ],
    out_shape=[
        jax.ShapeDtypeStruct((N_PAD, B), jnp.float32),
        jax.ShapeDtypeStruct((C16, B), jnp.float32),
        jax.ShapeDtypeStruct((C256, B), jnp.float32),
    ],
    scratch_shapes=[pltpu.VMEM((D_REP, B), jnp.float32)],
)


def _full16(v):
    return jnp.full((16,), v, jnp.int32)


def _merge16(bv, bi, nv, ni):
    sv, si = plsc.sort_key_val(nv, ni)
    rv = lax.rev(sv, (0,))
    ri = lax.rev(si, (0,))
    take = bv <= rv
    lov = jnp.where(take, bv, rv)
    loi = jnp.where(take, bi, ri)
    return plsc.sort_key_val(lov, loi)


def _scan_topk(nvec, load_fn):
    def body(i, carry):
        bv, bi, thr = carry
        nv, ni = load_fn(i)
        vmin = jnp.min(nv)

        def do(_):
            mv, mi = _merge16(bv, bi, nv, ni)
            return mv, mi, jnp.max(mv)

        def skip(_):
            return bv, bi, thr

        return lax.cond(vmin < thr, do, skip, None)

    bv0 = jnp.full((16,), INF, jnp.float32)
    bi0 = jnp.zeros((16,), jnp.int32)
    bv, bi, _ = lax.fori_loop(0, nvec, body, (bv0, bi0, jnp.float32(INF)))
    return bv, bi


def _rsqrt_newton(x):
    i = plsc.bitcast(x, jnp.int32)
    i = jnp.int32(0x5F3759DF) - lax.shift_right_arithmetic(i, jnp.int32(1))
    y = plsc.bitcast(i, jnp.float32)
    for _ in range(3):
        y = y * (1.5 - 0.5 * x * y * y)
    return y


def _sc_body(cm256_hbm, cm16_hbm, d2_hbm, act_hbm, out_hbm,
             slab, cidb, fidb, nidb, gidx, cand1, cand2, wbuf, outb,
             sem):
    cid = lax.axis_index("core")
    sid = lax.axis_index("subcore")
    wid = cid * 16 + sid
    base = wid * RPW
    iota = lax.iota(jnp.int32, 16)

    col0 = pl.multiple_of((wid // 4) * 128, 128)
    pltpu.sync_copy(cm256_hbm.at[:, pl.ds(col0, 128)], slab)
    lcol = (wid % 4) * RPW

    @pl.loop(0, RPW)
    def _(r):
        q = base + r

        def load_coarse(i):
            rows = i * 16 + iota
            v = plsc.load_gather(slab, [rows, _full16(lcol + r)])
            return v, rows

        _, ci = _scan_topk(C256 // 16, load_coarse)
        plsc.store_scatter(cidb, [_full16(r), iota], ci)

        @pl.loop(0, K)
        def _(j):
            cj = plsc.load_gather(cidb, [_full16(r), _full16(j)])
            idxv = (cj * 16 + iota) * B + q
            gidx[pl.ds(r * 256 + j * 16, 16)] = idxv

    @pl.loop(0, 8)
    def _(g):
        cs = []
        for u in range(8):
            off = (g * 8 + u) * 128
            cs.append(pltpu.async_copy(
                cm16_hbm.at[gidx.at[pl.ds(off, 128)]],
                cand1.at[pl.ds(off, 128)], sem))
        for c in cs:
            c.wait()

    @pl.loop(0, RPW)
    def _(r):
        q = base + r

        def load_fine(j):
            v = cand1[pl.ds(r * 256 + j * 16, 16)]
            cj = plsc.load_gather(cidb, [_full16(r), _full16(j)])
            return v, cj * 16 + iota

        _, fi = _scan_topk(K, load_fine)
        plsc.store_scatter(fidb, [_full16(r), iota], fi)

        @pl.loop(0, K)
        def _(j):
            fj = plsc.load_gather(fidb, [_full16(r), _full16(j)])
            idxv = (fj * 16 + iota) * B + q
            gidx[pl.ds(r * 256 + j * 16, 16)] = idxv

    @pl.loop(0, 8)
    def _(g):
        cs = []
        for u in range(8):
            off = (g * 8 + u) * 128
            cs.append(pltpu.async_copy(
                d2_hbm.at[gidx.at[pl.ds(off, 128)]],
                cand2.at[pl.ds(off, 128)], sem))
        for c in cs:
            c.wait()

    @pl.loop(0, RPW)
    def _(r):
        def load_elem(j):
            v = cand2[pl.ds(r * 256 + j * 16, 16)]
            fj = plsc.load_gather(fidb, [_full16(r), _full16(j)])
            return v, fj * 16 + iota

        dv, di = _scan_topk(K, load_elem)
        d2c = jnp.maximum(dv, 1e-12)
        d = d2c * _rsqrt_newton(d2c)
        dmin = jnp.min(d)
        e = jnp.exp(dmin - d)
        w = e / jnp.sum(e)
        plsc.store_scatter(wbuf, [_full16(r), iota], w)
        plsc.store_scatter(nidb, [_full16(r), iota], di)

        @pl.loop(0, K)
        def _(j):
            dj = plsc.load_gather(nidb, [_full16(r), _full16(j)])
            gidx[pl.ds(r * 256 + j * 16, 16)] = dj * 16 + iota

    @pl.loop(0, 8)
    def _(g):
        cs = []
        for u in range(8):
            off = (g * 8 + u) * 128
            cs.append(pltpu.async_copy(
                act_hbm.at[gidx.at[pl.ds(off, 128)]],
                cand1.at[pl.ds(off, 128)], sem))
        for c in cs:
            c.wait()

    @pl.loop(0, RPW)
    def _(r):
        def body(j, acc):
            a = cand1[pl.ds(r * 256 + j * 16, 16)]
            wj = plsc.load_gather(wbuf, [_full16(r), _full16(j)])
            return acc + wj * a

        acc = lax.fori_loop(0, K, body, jnp.zeros((16,), jnp.float32))
        plsc.store_scatter(outb, [_full16(r), iota], acc)

    pltpu.sync_copy(outb, out_hbm.at[pl.ds(base, RPW)])


@functools.cache
def _sc_call():
  cp = pltpu.CompilerParams()
  if "needs_layout_passes" in pltpu.CompilerParams.__dataclass_fields__:
    cp = dataclasses.replace(cp, needs_layout_passes=False)
  return functools.partial(
    pl.kernel,
    out_type=jax.ShapeDtypeStruct((B, 16), jnp.float32),
    mesh=plsc.VectorSubcoreMesh(core_axis_name="core",
                                subcore_axis_name="subcore"),
    compiler_params=cp,
    scratch_types=[
        pltpu.VMEM((C256, 128), jnp.float32),
        pltpu.VMEM((RPW, 16), jnp.int32),
        pltpu.VMEM((RPW, 16), jnp.int32),
        pltpu.VMEM((RPW, 16), jnp.int32),
        pltpu.VMEM((RPW * 256,), jnp.int32),
        pltpu.VMEM((RPW * 256,), jnp.float32),
        pltpu.VMEM((RPW * 256,), jnp.float32),
        pltpu.VMEM((RPW, 16), jnp.float32),
        pltpu.VMEM((RPW, 16), jnp.float32),
        pltpu.SemaphoreType.DMA,
    ],
  )(_sc_body)


def kernel(batch_images, W_enc, representations, actions):
    reps_p = jnp.pad(representations, ((0, N_PAD - N), (0, 0)),
                     constant_values=PAD_VAL)
    act_p = jnp.pad(actions, ((0, 0), (0, 16 - actions.shape[1])))
    d2t, cm16t, cm256t = _tc_call(batch_images, W_enc, reps_p)
    out = _sc_call()(cm256t, cm16t.reshape(-1), d2t.reshape(-1),
                     act_p.reshape(-1))
    return out[:, :7]
